# Initial kernel scaffold; baseline (speedup 1.0000x reference)
import jax
import jax.numpy as jnp
from jax.experimental import pallas as pl


def kernel(positions, cell, atomic_numbers, r_bins, q_bins):
    raise NotImplementedError("write your pallas kernel here")


# speedup vs baseline: 78.1281x; 78.1281x over previous
import functools

import jax
import jax.numpy as jnp
from jax import lax
from jax.experimental import pallas as pl
from jax.experimental.pallas import tpu as pltpu
from jax.experimental.pallas import tpu_sc as plsc

N = 3072
NBINS = 512
NQ = 256
CUTOFF = 3.9
B_SI = 4.1491
B_O = 5.803
FOURPI = 12.566370614359172

NC = 2
NS = 16
LN = 16
NW = NC * NS
RPW = N // NW
CH = 12
NCHUNK = RPW // CH
NJV = N // LN

BR = 256
BC = 128


def _s1_body(scal, prow, pcolt, out):
    L0, L1, L2 = scal[0], scal[1], scal[2]
    inv_dr = scal[3]

    def mi(d, L):
        h = L * 0.5
        return h - jnp.abs(jnp.abs(d) - h)

    xi = prow[:, 0:1]
    yi = prow[:, 1:2]
    zi = prow[:, 2:3]
    dx = mi(xi - pcolt[0:1, :], L0)
    dy = mi(yi - pcolt[1:2, :], L1)
    dz = mi(zi - pcolt[2:3, :], L2)
    dist = jnp.sqrt(dx * dx + dy * dy + dz * dz + 1e-12)
    row = lax.broadcasted_iota(jnp.int32, (BR, N), 0) + pl.program_id(0) * BR
    col = lax.broadcasted_iota(jnp.int32, (BR, N), 1)
    out[:, :] = jnp.where(row == col, jnp.float32(1e9), dist * inv_dr)


def _pairs_pos_idx(scal, positions, post):
    return pl.pallas_call(
        _s1_body,
        grid=(N // BR,),
        in_specs=[
            pl.BlockSpec(memory_space=pltpu.SMEM),
            pl.BlockSpec((BR, 3), lambda i: (i, 0)),
            pl.BlockSpec((3, N), lambda i: (0, 0)),
        ],
        out_specs=pl.BlockSpec((BR, N), lambda i: (i, 0)),
        out_shape=jax.ShapeDtypeStruct((N, N), jnp.float32),
    )(scal, positions, post)


@functools.cache
def _get_sc_hist():
    mesh = plsc.VectorSubcoreMesh(core_axis_name="c", subcore_axis_name="s")
    return functools.partial(
        pl.kernel,
        mesh=mesh,
        compiler_params=pltpu.CompilerParams(
            use_tc_tiling_on_sc=False, needs_layout_passes=False),
        out_type=jax.ShapeDtypeStruct((NW, 4, LN, NBINS + 1), jnp.float32),
        scratch_types=[
            pltpu.VMEM((2, CH, N), jnp.float32),
            pltpu.VMEM((4, LN, NBINS + 1), jnp.float32),
            pltpu.SemaphoreType.DMA,
            pltpu.SemaphoreType.DMA,
        ],
    )(_sc_hist_body)


def _sc_hist_body(p_hbm, z_hbm, out_hbm, buf, hist, sem0, sem1):
    cid = lax.axis_index("c")
    sid = lax.axis_index("s")
    wid = sid * NC + cid
    row0 = wid * RPW
    pltpu.sync_copy(z_hbm, hist)
    lane = lax.iota(jnp.int32, LN)
    tj = [jnp.where(lax.rem(lane + c, 3) == 0, 0, 1) for c in range(3)]
    sems = (sem0, sem1)
    copies = [None, None]

    def issue(ci):
        slot = ci % 2
        copies[slot] = pltpu.async_copy(
            p_hbm.at[pl.ds(row0 + ci * CH, CH)], buf.at[slot], sems[slot])

    issue(0)
    for ci in range(NCHUNK):
        if ci + 1 < NCHUNK:
            issue(ci + 1)
        copies[ci % 2].wait()
        slot = ci % 2
        rbase = row0 + ci * CH

        def row_body(r, carry, *, slot=slot, rbase=rbase):
            i = rbase + r
            ti2 = jnp.where(lax.rem(i, 3) == 0, 0, 2)
            tvecs = [tj[c] + ti2 for c in range(3)]

            def j_body(t, inner):
                for u in range(3):
                    c = t * 3 + u
                    j0 = pl.multiple_of(c * LN, LN)
                    p = buf[slot, r, pl.ds(j0, LN)]
                    lo = p.astype(jnp.int32)
                    frac = p - lo.astype(jnp.float32)
                    w0 = 1.0 - frac
                    valid = p < float(NBINS)
                    hi = lo + 1
                    tv = tvecs[u]
                    plsc.addupdate_scatter(hist, [tv, lane, lo], w0, mask=valid)
                    plsc.addupdate_scatter(hist, [tv, lane, hi], frac, mask=valid)
                return inner

            return lax.fori_loop(0, NJV // 3, j_body, carry, unroll=4)

        lax.fori_loop(0, CH, row_body, jnp.int32(0))
    pltpu.sync_copy(hist, out_hbm.at[wid])


def _s3_body(scal, cent, pcolt, omask, out):
    L0, L1, L2 = scal[0], scal[1], scal[2]

    def mi(d, L):
        h = L * 0.5
        return jnp.where(d > h, d - L, jnp.where(d < -h, d + L, d))

    def mag(d, L):
        h = L * 0.5
        return h - jnp.abs(jnp.abs(d) - h)

    dx = mag(cent[:, 0:1] - pcolt[0:1, :], L0)
    dy = mag(cent[:, 1:2] - pcolt[1:2, :], L1)
    dz = mag(cent[:, 2:3] - pcolt[2:3, :], L2)
    d = jnp.sqrt(dx * dx + dy * dy + dz * dz + 1e-12)
    valid = (omask[0:1, :] > 0.0) & (d < CUTOFF)
    cnt = jnp.sum(valid.astype(jnp.float32), axis=1, keepdims=True)
    inf = jnp.float32(jnp.inf)
    dm = jnp.where(valid, d, inf)
    jcol = lax.broadcasted_iota(jnp.int32, (BC, N), 1)
    us = []
    for _ in range(4):
        m = jnp.min(dm, axis=1, keepdims=True)
        idx = jnp.min(jnp.where(dm == m, jcol, N), axis=1, keepdims=True)
        selb = jcol == idx
        dm = jnp.where(selb, inf, dm)
        sf = selb.astype(jnp.float32)
        pj = lax.dot_general(sf, pcolt[:, :], (((1,), (1,)), ((), ())),
                             preferred_element_type=jnp.float32)
        vx = mi(pj[:, 0:1] - cent[:, 0:1], L0)
        vy = mi(pj[:, 1:2] - cent[:, 1:2], L1)
        vz = mi(pj[:, 2:3] - cent[:, 2:3], L2)
        nrm = jnp.sqrt(vx * vx + vy * vy + vz * vz)
        inv = 1.0 / jnp.maximum(nrm, 1e-12)
        us.append((vx * inv, vy * inv, vz * inv))
    acc = jnp.zeros((BC, 1), jnp.float32)
    for a in range(4):
        for b in range(a + 1, 4):
            ua, ub = us[a], us[b]
            c = ua[0] * ub[0] + ua[1] * ub[1] + ua[2] * ub[2]
            c = jnp.clip(c, -1.0, 1.0)
            acc = acc + (c + 1.0 / 3.0) ** 2
    q = 1.0 - 0.375 * acc
    out[:, :] = jnp.where(cnt >= 4.0, q, 0.0)


def _qtet(scal, cent_pos, post, omask):
    ncent = (N + 2) // 3
    return pl.pallas_call(
        _s3_body,
        grid=(ncent // BC,),
        in_specs=[
            pl.BlockSpec(memory_space=pltpu.SMEM),
            pl.BlockSpec((BC, 3), lambda i: (i, 0)),
            pl.BlockSpec((3, N), lambda i: (0, 0)),
            pl.BlockSpec((1, N), lambda i: (0, 0)),
        ],
        out_specs=pl.BlockSpec((BC, 1), lambda i: (i, 0)),
        out_shape=jax.ShapeDtypeStruct((ncent, 1), jnp.float32),
    )(scal, cent_pos, post, omask)


def _s4_body(scal, parts, rmid, qcol, gout, tout, sout):
    rho = scal[0]
    dr = scal[1]
    fn = scal[2]
    s_sisi = scal[3]
    s_sio = scal[4]
    s_oo = scal[5]
    rt = (lax.broadcasted_iota(jnp.int32, (NW * 4 * LN, 1), 0) // LN) % 4
    scale = jnp.where(rt == 0, s_sisi,
                      jnp.where(rt == 3, s_oo, s_sio))
    hist513 = jnp.sum(scale * parts[:, :], axis=0, keepdims=True)
    ci = lax.broadcasted_iota(jnp.int32, (1, NBINS), 1)
    hist = hist513[:, :NBINS] + jnp.where(
        ci == NBINS - 1, hist513[:, NBINS:NBINS + 1], 0.0)
    rm = rmid[:, :]
    shell = (FOURPI * dr) * rm * rm
    g = hist / (fn * rho * shell)
    gm1 = g - 1.0
    G = (FOURPI * rho) * rm * gm1
    T = G + (FOURPI * rho) * rm
    Q = qcol[:, :]
    integ = jnp.sum((rm * dr) * gm1 * jnp.sin(Q * rm), axis=1, keepdims=True)
    S = 1.0 + (FOURPI * rho) / Q * integ
    gout[:, :] = G
    tout[:, :] = T
    sout[:, :] = S


def _postproc(scal, parts, rmid, qcol):
    return pl.pallas_call(
        _s4_body,
        in_specs=[
            pl.BlockSpec(memory_space=pltpu.SMEM),
            pl.BlockSpec((NW * 4 * LN, NBINS + 1), lambda: (0, 0)),
            pl.BlockSpec((1, NBINS), lambda: (0, 0)),
            pl.BlockSpec((NQ, 1), lambda: (0, 0)),
        ],
        out_specs=[
            pl.BlockSpec((1, NBINS), lambda: (0, 0)),
            pl.BlockSpec((1, NBINS), lambda: (0, 0)),
            pl.BlockSpec((NQ, 1), lambda: (0, 0)),
        ],
        out_shape=[
            jax.ShapeDtypeStruct((1, NBINS), jnp.float32),
            jax.ShapeDtypeStruct((1, NBINS), jnp.float32),
            jax.ShapeDtypeStruct((NQ, 1), jnp.float32),
        ],
    )(scal, parts, rmid, qcol)


def kernel(positions, cell, atomic_numbers, r_bins, q_bins):
    positions = positions.astype(jnp.float32)
    L = jnp.diagonal(cell).astype(jnp.float32)
    dr = (r_bins[1] - r_bins[0]).astype(jnp.float32)
    b = jnp.where(atomic_numbers == 14, B_SI, B_O).astype(jnp.float32)
    bm2 = jnp.mean(b) ** 2
    scal1 = jnp.stack([L[0], L[1], L[2], 1.0 / dr])
    post = positions.T

    p_mat = _pairs_pos_idx(scal1, positions, post)

    zeros = jnp.zeros((4, LN, NBINS + 1), jnp.float32)
    parts = _get_sc_hist()(p_mat, zeros)

    cent_pos = positions[::3]
    omask = (atomic_numbers[None, :] != 14).astype(jnp.float32)
    scal3 = jnp.stack([L[0], L[1], L[2], 0.0])
    q2d = _qtet(scal3, cent_pos, post, omask)

    vol = L[0] * L[1] * L[2]
    rho = jnp.float32(N) / vol
    r_mid = (0.5 * (r_bins[:-1] + r_bins[1:])).astype(jnp.float32)
    bsi = jnp.float32(B_SI)
    bo = jnp.float32(B_O)
    scal4 = jnp.stack([rho, dr, jnp.float32(N),
                       bsi * bsi / bm2, bsi * bo / bm2, bo * bo / bm2])
    G2, T2, S2 = _postproc(scal4, parts.reshape(NW * 4 * LN, NBINS + 1),
                           r_mid.reshape(1, NBINS),
                           q_bins.astype(jnp.float32).reshape(NQ, 1))
    return (G2.reshape(NBINS), T2.reshape(NBINS), S2.reshape(NQ),
            q2d.reshape((N + 2) // 3))
